# Initial kernel scaffold; baseline (speedup 1.0000x reference)
import jax
import jax.numpy as jnp
from jax.experimental import pallas as pl


def kernel(tensor, weights):
    raise NotImplementedError("write your pallas kernel here")


# speedup vs baseline: 1.8334x; 1.8334x over previous
import functools

import jax
import jax.numpy as jnp
from jax import lax
from jax.experimental import pallas as pl
from jax.experimental.pallas import tpu as pltpu
from jax.experimental.pallas import tpu_sc as plsc

IN_DIM = 1000000
OUT_DIM = 64
BATCH = 16384
HIST = 50

NC = 2
NS = 16
NW = NC * NS

B = BATCH * HIST
BPW = B // NW
IDX_ROW = 128
N_IDX_ROWS = BPW // IDX_ROW
CHUNK = 512
G_PER_CHUNK = CHUNK // IDX_ROW
NCHUNK = BPW // CHUNK


def _gather_body(table_hbm, idx_hbm, out_hbm, idx_v, rows_v, gsem):
    wid = lax.axis_index("s") * NC + lax.axis_index("c")
    base = wid * BPW
    pltpu.sync_copy(idx_hbm.at[wid], idx_v)

    @pl.loop(0, NCHUNK)
    def _chunk(ci):
        copies = []
        for g in range(G_PER_CHUNK):
            row = ci * G_PER_CHUNK + g
            copies.append(
                pltpu.async_copy(
                    table_hbm.at[idx_v.at[row]],
                    rows_v.at[pl.ds(g * IDX_ROW, IDX_ROW)],
                    gsem,
                )
            )
        for c in copies:
            c.wait()
        pltpu.sync_copy(rows_v, out_hbm.at[pl.ds(base + ci * CHUNK, CHUNK)])


@jax.jit
def _embedding_gather(idx, weights):
    mesh = plsc.VectorSubcoreMesh(
        core_axis_name="c", subcore_axis_name="s",
        num_cores=NC, num_subcores=NS,
    )
    run = functools.partial(
        pl.kernel,
        mesh=mesh,
        out_type=jax.ShapeDtypeStruct((B, OUT_DIM), jnp.float32),
        scratch_types=[
            pltpu.VMEM((N_IDX_ROWS, IDX_ROW), jnp.int32),
            pltpu.VMEM((CHUNK, OUT_DIM), jnp.float32),
            pltpu.SemaphoreType.DMA,
        ],
        compiler_params=pltpu.CompilerParams(use_tc_tiling_on_sc=False),
    )(_gather_body)
    return run(weights, idx)


def kernel(tensor, weights):
    idx = tensor.reshape(NW, N_IDX_ROWS, IDX_ROW).astype(jnp.int32)
    out = _embedding_gather(idx, weights)
    return out.reshape(BATCH, HIST, OUT_DIM)


# speedup vs baseline: 1.8778x; 1.0242x over previous
import functools

import jax
import jax.numpy as jnp
from jax import lax
from jax.experimental import pallas as pl
from jax.experimental.pallas import tpu as pltpu
from jax.experimental.pallas import tpu_sc as plsc

IN_DIM = 1000000
OUT_DIM = 64
BATCH = 16384
HIST = 50

NC = 2
NS = 16
NW = NC * NS

B = BATCH * HIST
BPW = B // NW
IDX_ROW = 128
N_IDX_ROWS = BPW // IDX_ROW
CHUNK = 256
G_PER_CHUNK = CHUNK // IDX_ROW
NCHUNK = BPW // CHUNK
NBUF = 5
LOOKAHEAD = NBUF - 1


def _fire_gathers(table_hbm, idx_v, rows_v, gsem, chunk, slot):
    for g in range(G_PER_CHUNK):
        pltpu.async_copy(
            table_hbm.at[idx_v.at[chunk * G_PER_CHUNK + g]],
            rows_v.at[slot, pl.ds(g * IDX_ROW, IDX_ROW)],
            gsem.at[slot],
        )


def _gather_body(table_hbm, idx_hbm, out_hbm, idx_v, rows_v, gsem, osem):
    wid = lax.axis_index("s") * NC + lax.axis_index("c")
    base = wid * BPW
    pltpu.sync_copy(idx_hbm.at[wid], idx_v)

    for c in range(LOOKAHEAD):
        _fire_gathers(table_hbm, idx_v, rows_v, gsem, c, c)

    @pl.loop(0, NCHUNK, step=NBUF)
    def _outer(ci0):
        for b in range(NBUF):
            ci = ci0 + b
            sg = (b + LOOKAHEAD) % NBUF
            ahead_ok = ci + LOOKAHEAD < NCHUNK

            @pl.when(jnp.logical_and(ahead_ok, ci >= 1))
            def _wait_out():
                pltpu.make_async_copy(
                    table_hbm.at[pl.ds(0, CHUNK)],
                    rows_v.at[sg],
                    osem.at[sg],
                ).wait()

            @pl.when(ahead_ok)
            def _refill():
                _fire_gathers(table_hbm, idx_v, rows_v, gsem,
                              ci + LOOKAHEAD, sg)

            pltpu.make_async_copy(
                table_hbm.at[pl.ds(0, CHUNK)],
                rows_v.at[b],
                gsem.at[b],
            ).wait()
            pltpu.async_copy(
                rows_v.at[b],
                out_hbm.at[pl.ds(base + ci * CHUNK, CHUNK)],
                osem.at[b],
            )

    for b in range(NBUF):
        pltpu.make_async_copy(
            table_hbm.at[pl.ds(0, CHUNK)],
            rows_v.at[b],
            osem.at[b],
        ).wait()


@jax.jit
def _embedding_gather(idx, weights):
    mesh = plsc.VectorSubcoreMesh(
        core_axis_name="c", subcore_axis_name="s",
        num_cores=NC, num_subcores=NS,
    )
    run = functools.partial(
        pl.kernel,
        mesh=mesh,
        out_type=jax.ShapeDtypeStruct((B, OUT_DIM), jnp.float32),
        scratch_types=[
            pltpu.VMEM((N_IDX_ROWS, IDX_ROW), jnp.int32),
            pltpu.VMEM((NBUF, CHUNK, OUT_DIM), jnp.float32),
            pltpu.SemaphoreType.DMA((NBUF,)),
            pltpu.SemaphoreType.DMA((NBUF,)),
        ],
        compiler_params=pltpu.CompilerParams(use_tc_tiling_on_sc=False),
    )(_gather_body)
    return run(weights, idx)


def kernel(tensor, weights):
    idx = tensor.reshape(NW, N_IDX_ROWS, IDX_ROW).astype(jnp.int32)
    out = _embedding_gather(idx, weights)
    return out.reshape(BATCH, HIST, OUT_DIM)
